# baseline (speedup 1.0000x reference)
import dataclasses
import functools

import jax
import jax.numpy as jnp
from jax import lax
from jax.experimental import pallas as pl
from jax.experimental.pallas import tpu as pltpu
from jax.experimental.pallas import tpu_sc as plsc

_N = 10000
_E = 320000
_D = 128
_NC = 2
_NS = 16
_NW = _NC * _NS
_C = 128
_NCHUNK = 2560
_EP = _NCHUNK * _C
_NP = 10240
_HR = _NP // _D
_RPT = _NP // _NS


def _sc_compiler_params():
    cp = pltpu.CompilerParams()
    if "needs_layout_passes" in pltpu.CompilerParams.__dataclass_fields__:
        cp = dataclasses.replace(cp, needs_layout_passes=False)
    return cp


def _sc_aggregate(x, src, dst):
    mesh = plsc.VectorSubcoreMesh(core_axis_name="c", subcore_axis_name="s")

    @functools.partial(
        pl.kernel,
        mesh=mesh,
        compiler_params=_sc_compiler_params(),
        out_type=(
            jax.ShapeDtypeStruct((_NC, _NP, _D), jnp.float32),
            jax.ShapeDtypeStruct((_NC, _HR, _D), jnp.float32),
        ),
        scratch_types=[
            pltpu.VMEM((_C,), jnp.int32),
            pltpu.VMEM((_C,), jnp.int32),
            pltpu.VMEM((_C,), jnp.int32),
            pltpu.VMEM((_C,), jnp.int32),
            pltpu.VMEM((_HR,), jnp.int32),
            pltpu.VMEM((_C, _D), jnp.float32),
            pltpu.VMEM((_C, _D), jnp.float32),
            pltpu.VMEM((_HR, _D), jnp.float32),
            pltpu.VMEM_SHARED((_NP, _D), jnp.float32),
            pltpu.VMEM_SHARED((_HR, _D), jnp.float32),
            pltpu.SemaphoreType.DMA,
            pltpu.SemaphoreType.DMA,
            pltpu.SemaphoreType.DMA,
            pltpu.SemaphoreType.DMA,
            pltpu.SemaphoreType.DMA,
            pltpu.SemaphoreType.DMA,
        ],
    )
    def agg(x_hbm, src_hbm, dst_hbm,
            sum_hbm, cnt_hbm,
            src_a, src_b, dst_a, dst_b, iota_v, rows_a, rows_b, hist_v,
            acc_sh, cnt_sh,
            isa, isb, gsa, gsb, ssa, ssb):
        cid = lax.axis_index("c")
        sid = lax.axis_index("s")
        wid = sid * _NC + cid
        row0 = sid * _RPT

        zeros16 = jnp.zeros((16,), jnp.float32)
        iota16 = jnp.arange(16, dtype=jnp.int32)

        @pl.loop(0, _HR // 16)
        def _(r):
            iota_v[pl.ds(r * 16, 16)] = iota16 + r * 16

        @pl.loop(0, _HR)
        def _(r):
            @pl.loop(0, _D, step=16)
            def _(c):
                hist_v[r, pl.ds(c, 16)] = zeros16

        @pl.loop(0, _C)
        def _(r):
            @pl.loop(0, _D, step=16)
            def _(c):
                rows_a[r, pl.ds(c, 16)] = zeros16

        @pl.loop(0, _RPT // _C)
        def _(k):
            pltpu.sync_copy(rows_a, acc_sh.at[pl.ds(row0 + k * _C, _C)])

        @pl.when(sid == 0)
        def _():
            pltpu.sync_copy(hist_v, cnt_sh)

        plsc.subcore_barrier()

        ones16 = jnp.ones((16,), jnp.float32)

        def hist_chunk(dst_ref):
            @pl.loop(0, _C, step=16)
            def _(k):
                idx = dst_ref[pl.ds(k, 16)]
                plsc.addupdate_scatter(hist_v, [idx >> 7, idx & 127], ones16)

        @pl.loop(wid, _NCHUNK, step=_NW)
        def _(g):
            base = g * _C
            pltpu.sync_copy(src_hbm.at[pl.ds(base, _C)], src_a)
            pltpu.sync_copy(dst_hbm.at[pl.ds(base, _C)], dst_a)
            pltpu.sync_copy(x_hbm.at[src_a], rows_a)
            pltpu.sync_copy(rows_a, acc_sh.at[dst_a], add=True)
            hist_chunk(dst_a)

        pltpu.sync_copy(hist_v, cnt_sh.at[iota_v], add=True)
        plsc.subcore_barrier()

        pltpu.sync_copy(acc_sh.at[pl.ds(row0, _RPT)],
                        sum_hbm.at[cid, pl.ds(row0, _RPT)])

        @pl.when(sid == 0)
        def _():
            pltpu.sync_copy(cnt_sh, cnt_hbm.at[cid])

    return agg(x, src, dst)


def _tc_finish(parts, cnt, x, W_l, b_l, W_r, W_fc, b_fc):
    def body(pp, cc, xr, wl, bl, wr, wfc, bfc, out):
        p = pp[0, :_N, :] + pp[1, :_N, :]
        mean = p / jnp.maximum(cc[...], 1.0)
        h = (jnp.dot(mean, wl[...], preferred_element_type=jnp.float32)
             + jnp.dot(xr[...], wr[...], preferred_element_type=jnp.float32)
             + bl[...])
        h = jnp.maximum(h, 0.0)
        out[...] = jnp.dot(h, wfc[...], preferred_element_type=jnp.float32) + bfc[...]

    return pl.pallas_call(
        body,
        out_shape=jax.ShapeDtypeStruct((_N, 1), jnp.float32),
    )(parts, cnt, x, W_l, b_l, W_r, W_fc, b_fc)


def kernel(x, edge_index, W_l, b_l, W_r, W_fc, b_fc):
    npad = _EP - _E
    srcp = jnp.concatenate([edge_index[0], jnp.zeros((npad,), jnp.int32)])
    dstp = jnp.concatenate(
        [edge_index[1], _N + (jnp.arange(npad, dtype=jnp.int32) % (_NP - _N))])
    parts, cnts = _sc_aggregate(x, srcp, dstp)
    cnt = (cnts[0] + cnts[1]).reshape(_NP, 1)[:_N]
    return _tc_finish(parts, cnt, x, W_l, b_l[None, :], W_r, W_fc, b_fc[None, :])


# speedup vs baseline: 1.1506x; 1.1506x over previous
import dataclasses
import functools

import jax
import jax.numpy as jnp
from jax import lax
from jax.experimental import pallas as pl
from jax.experimental.pallas import tpu as pltpu
from jax.experimental.pallas import tpu_sc as plsc

_N = 10000
_E = 320000
_D = 128
_NC = 2
_NS = 16
_NW = _NC * _NS
_C = 128
_NCHUNK = 2560
_EP = _NCHUNK * _C
_NP = 10240
_HR = _NP // _D
_RPT = _NP // _NS


def _sc_compiler_params():
    cp = pltpu.CompilerParams()
    if "needs_layout_passes" in pltpu.CompilerParams.__dataclass_fields__:
        cp = dataclasses.replace(cp, needs_layout_passes=False)
    return cp


def _sc_aggregate(x, src, dst, zfeat):
    mesh = plsc.VectorSubcoreMesh(core_axis_name="c", subcore_axis_name="s")

    @functools.partial(
        pl.kernel,
        mesh=mesh,
        compiler_params=_sc_compiler_params(),
        out_type=(
            jax.ShapeDtypeStruct((_NC, _NP, _D), jnp.float32),
            jax.ShapeDtypeStruct((_NC, _HR, _D), jnp.float32),
        ),
        scratch_types=[
            pltpu.VMEM((_C,), jnp.int32),
            pltpu.VMEM((_C,), jnp.int32),
            pltpu.VMEM((_C,), jnp.int32),
            pltpu.VMEM((_C,), jnp.int32),
            pltpu.VMEM((_HR,), jnp.int32),
            pltpu.VMEM((_C, _D), jnp.float32),
            pltpu.VMEM((_C, _D), jnp.float32),
            pltpu.VMEM((_HR, _D), jnp.float32),
            pltpu.VMEM_SHARED((_NP, _D), jnp.float32),
            pltpu.VMEM_SHARED((_HR, _D), jnp.float32),
            pltpu.SemaphoreType.DMA,
            pltpu.SemaphoreType.DMA,
            pltpu.SemaphoreType.DMA,
            pltpu.SemaphoreType.DMA,
            pltpu.SemaphoreType.DMA,
            pltpu.SemaphoreType.DMA,
        ],
    )
    def agg(x_hbm, src_hbm, dst_hbm, zf_hbm,
            sum_hbm, cnt_hbm,
            src_a, src_b, dst_a, dst_b, iota_v, rows_a, rows_b, hist_v,
            acc_sh, cnt_sh,
            isa, isb, gsa, gsb, ssa, ssb):
        cid = lax.axis_index("c")
        sid = lax.axis_index("s")
        wid = sid * _NC + cid
        row0 = sid * _RPT

        zeros16 = jnp.zeros((16,), jnp.float32)
        iota16 = jnp.arange(16, dtype=jnp.int32)

        @pl.loop(0, _HR // 16)
        def _(r):
            iota_v[pl.ds(r * 16, 16)] = iota16 + r * 16

        @pl.loop(0, _HR)
        def _(r):
            @pl.loop(0, _D, step=16)
            def _(c):
                hist_v[r, pl.ds(c, 16)] = zeros16

        pltpu.sync_copy(zf_hbm, acc_sh.at[pl.ds(row0, _RPT)])

        @pl.when(sid == 0)
        def _():
            pltpu.sync_copy(zf_hbm.at[pl.ds(0, _HR)], cnt_sh)

        plsc.subcore_barrier()

        ones16 = jnp.ones((16,), jnp.float32)

        def hist_chunk(dst_ref):
            @pl.loop(0, _C, step=16)
            def _(k):
                idx = dst_ref[pl.ds(k, 16)]
                plsc.addupdate_scatter(hist_v, [idx >> 7, idx & 127], ones16)

        @pl.loop(0, _NCHUNK // (2 * _NW))
        def _(q):
            g0 = wid + q * (2 * _NW)
            g1 = g0 + _NW
            b0 = g0 * _C
            b1 = g1 * _C
            hia0 = pltpu.async_copy(src_hbm.at[pl.ds(b0, _C)], src_a, isa)
            hia1 = pltpu.async_copy(dst_hbm.at[pl.ds(b0, _C)], dst_a, isa)
            hib0 = pltpu.async_copy(src_hbm.at[pl.ds(b1, _C)], src_b, isb)
            hib1 = pltpu.async_copy(dst_hbm.at[pl.ds(b1, _C)], dst_b, isb)
            hia0.wait()
            hia1.wait()
            hga = pltpu.async_copy(x_hbm.at[src_a], rows_a, gsa)
            hib0.wait()
            hib1.wait()
            hgb = pltpu.async_copy(x_hbm.at[src_b], rows_b, gsb)
            hga.wait()
            hsa = pltpu.async_copy(rows_a, acc_sh.at[dst_a], ssa, add=True)
            hist_chunk(dst_a)
            hgb.wait()
            hsb = pltpu.async_copy(rows_b, acc_sh.at[dst_b], ssb, add=True)
            hist_chunk(dst_b)
            hsa.wait()
            hsb.wait()

        pltpu.sync_copy(hist_v, cnt_sh.at[iota_v], add=True)
        plsc.subcore_barrier()

        pltpu.sync_copy(acc_sh.at[pl.ds(row0, _RPT)],
                        sum_hbm.at[cid, pl.ds(row0, _RPT)])

        @pl.when(sid == 0)
        def _():
            pltpu.sync_copy(cnt_sh, cnt_hbm.at[cid])

    return agg(x, src, dst, zfeat)


def _tc_finish(parts, cnt, x, W_l, b_l, W_r, W_fc, b_fc):
    def body(pp, cc, xr, wl, bl, wr, wfc, bfc, out):
        p = pp[0, :_N, :] + pp[1, :_N, :]
        mean = p / jnp.maximum(cc[...], 1.0)
        h = (jnp.dot(mean, wl[...], preferred_element_type=jnp.float32)
             + jnp.dot(xr[...], wr[...], preferred_element_type=jnp.float32)
             + bl[...])
        h = jnp.maximum(h, 0.0)
        out[...] = jnp.dot(h, wfc[...], preferred_element_type=jnp.float32) + bfc[...]

    return pl.pallas_call(
        body,
        out_shape=jax.ShapeDtypeStruct((_N, 1), jnp.float32),
    )(parts, cnt, x, W_l, b_l, W_r, W_fc, b_fc)


def kernel(x, edge_index, W_l, b_l, W_r, W_fc, b_fc):
    npad = _EP - _E
    srcp = jnp.concatenate([edge_index[0], jnp.zeros((npad,), jnp.int32)])
    dstp = jnp.concatenate(
        [edge_index[1], _N + (jnp.arange(npad, dtype=jnp.int32) % (_NP - _N))])
    zfeat = jnp.zeros((_RPT, _D), jnp.float32)
    parts, cnts = _sc_aggregate(x, srcp, dstp, zfeat)
    cnt = (cnts[0] + cnts[1]).reshape(_NP, 1)[:_N]
    return _tc_finish(parts, cnt, x, W_l, b_l[None, :], W_r, W_fc, b_fc[None, :])


# speedup vs baseline: 2.6417x; 2.2959x over previous
import dataclasses
import functools

import jax
import jax.numpy as jnp
from jax import lax
from jax.experimental import pallas as pl
from jax.experimental.pallas import tpu as pltpu
from jax.experimental.pallas import tpu_sc as plsc

_N = 10000
_E = 320000
_D = 128
_NC = 2
_NS = 16
_NW = _NC * _NS
_C = 128
_NCHUNK = _E // _C
_NP = 10240
_HR = _NP // _D
_RPT = _NP // _NS


def _sc_compiler_params():
    cp = pltpu.CompilerParams()
    if "needs_layout_passes" in pltpu.CompilerParams.__dataclass_fields__:
        cp = dataclasses.replace(cp, needs_layout_passes=False)
    return cp


def _sc_aggregate(x, src, dst, zfeat):
    mesh = plsc.VectorSubcoreMesh(core_axis_name="c", subcore_axis_name="s")

    @functools.partial(
        pl.kernel,
        mesh=mesh,
        compiler_params=_sc_compiler_params(),
        out_type=(
            jax.ShapeDtypeStruct((_NC, _NP, _D), jnp.float32),
            jax.ShapeDtypeStruct((_NC, _HR, _D), jnp.float32),
        ),
        scratch_types=[
            pltpu.VMEM((_C,), jnp.int32),
            pltpu.VMEM((_C,), jnp.int32),
            pltpu.VMEM((_C,), jnp.int32),
            pltpu.VMEM((_C,), jnp.int32),
            pltpu.VMEM((_HR,), jnp.int32),
            pltpu.VMEM((_C, _D), jnp.float32),
            pltpu.VMEM((_C, _D), jnp.float32),
            pltpu.VMEM((_HR, _D), jnp.float32),
            pltpu.VMEM_SHARED((_NP, _D), jnp.float32),
            pltpu.VMEM_SHARED((_HR, _D), jnp.float32),
            pltpu.SemaphoreType.DMA,
            pltpu.SemaphoreType.DMA,
            pltpu.SemaphoreType.DMA,
            pltpu.SemaphoreType.DMA,
            pltpu.SemaphoreType.DMA,
            pltpu.SemaphoreType.DMA,
        ],
    )
    def agg(x_hbm, src_hbm, dst_hbm, zf_hbm,
            sum_hbm, cnt_hbm,
            src_a, src_b, dst_a, dst_b, iota_v, rows_a, rows_b, hist_v,
            acc_sh, cnt_sh,
            isa, isb, gsa, gsb, ssa, ssb):
        cid = lax.axis_index("c")
        sid = lax.axis_index("s")
        wid = sid * _NC + cid
        row0 = sid * _RPT

        zeros16 = jnp.zeros((16,), jnp.float32)
        iota16 = jnp.arange(16, dtype=jnp.int32)

        @pl.loop(0, _HR // 16)
        def _(r):
            iota_v[pl.ds(r * 16, 16)] = iota16 + r * 16

        @pl.loop(0, _HR)
        def _(r):
            @pl.loop(0, _D, step=16)
            def _(c):
                hist_v[r, pl.ds(c, 16)] = zeros16

        pltpu.sync_copy(zf_hbm, acc_sh.at[pl.ds(row0, _RPT)])

        @pl.when(sid == 0)
        def _():
            pltpu.sync_copy(zf_hbm.at[pl.ds(0, _HR)], cnt_sh)

        plsc.subcore_barrier()

        ones16 = jnp.ones((16,), jnp.float32)

        def hist_chunk(dst_ref):
            @pl.loop(0, _C, step=16)
            def _(k):
                idx = dst_ref[pl.ds(k, 16)]
                plsc.addupdate_scatter(hist_v, [idx >> 7, idx & 127], ones16)

        @pl.loop(0, _NCHUNK // (2 * _NW))
        def _(q):
            g0 = wid + q * (2 * _NW)
            g1 = g0 + _NW
            b0 = g0 * _C
            b1 = g1 * _C
            hia0 = pltpu.async_copy(src_hbm.at[pl.ds(b0, _C)], src_a, isa)
            hia1 = pltpu.async_copy(dst_hbm.at[pl.ds(b0, _C)], dst_a, isa)
            hib0 = pltpu.async_copy(src_hbm.at[pl.ds(b1, _C)], src_b, isb)
            hib1 = pltpu.async_copy(dst_hbm.at[pl.ds(b1, _C)], dst_b, isb)
            hia0.wait()
            hia1.wait()
            hga = pltpu.async_copy(x_hbm.at[src_a], rows_a, gsa)
            hib0.wait()
            hib1.wait()
            hgb = pltpu.async_copy(x_hbm.at[src_b], rows_b, gsb)
            hga.wait()
            hsa = pltpu.async_copy(rows_a, acc_sh.at[dst_a], ssa, add=True)
            hist_chunk(dst_a)
            hgb.wait()
            hsb = pltpu.async_copy(rows_b, acc_sh.at[dst_b], ssb, add=True)
            hist_chunk(dst_b)
            hsa.wait()
            hsb.wait()

        @pl.when(wid < _NCHUNK - (_NCHUNK // (2 * _NW)) * 2 * _NW)
        def _():
            base = ((_NCHUNK // (2 * _NW)) * 2 * _NW + wid) * _C
            pltpu.sync_copy(src_hbm.at[pl.ds(base, _C)], src_a)
            pltpu.sync_copy(dst_hbm.at[pl.ds(base, _C)], dst_a)
            pltpu.sync_copy(x_hbm.at[src_a], rows_a)
            pltpu.sync_copy(rows_a, acc_sh.at[dst_a], add=True)
            hist_chunk(dst_a)

        pltpu.sync_copy(hist_v, cnt_sh.at[iota_v], add=True)
        plsc.subcore_barrier()

        pltpu.sync_copy(acc_sh.at[pl.ds(row0, _RPT)],
                        sum_hbm.at[cid, pl.ds(row0, _RPT)])

        @pl.when(sid == 0)
        def _():
            pltpu.sync_copy(cnt_sh, cnt_hbm.at[cid])

    return agg(x, src, dst, zfeat)


def _tc_finish(parts, cnt, x, W_l, b_l, W_r, W_fc, b_fc):
    def body(pp, cc, xr, wl, bl, wr, wfc, bfc, out):
        p = pp[0, :_N, :] + pp[1, :_N, :]
        mean = p / jnp.maximum(cc[...], 1.0)
        h = (jnp.dot(mean, wl[...], preferred_element_type=jnp.float32)
             + jnp.dot(xr[...], wr[...], preferred_element_type=jnp.float32)
             + bl[...])
        h = jnp.maximum(h, 0.0)
        out[...] = jnp.dot(h, wfc[...], preferred_element_type=jnp.float32) + bfc[...]

    return pl.pallas_call(
        body,
        out_shape=jax.ShapeDtypeStruct((_N, 1), jnp.float32),
    )(parts, cnt, x, W_l, b_l, W_r, W_fc, b_fc)


def kernel(x, edge_index, W_l, b_l, W_r, W_fc, b_fc):
    zfeat = jnp.zeros((_RPT, _D), jnp.float32)
    parts, cnts = _sc_aggregate(x, edge_index[0], edge_index[1], zfeat)
    cnt = (cnts[0] + cnts[1]).reshape(_NP, 1)[:_N]
    return _tc_finish(parts, cnt, x, W_l, b_l[None, :], W_r, W_fc, b_fc[None, :])
